# baseline (speedup 1.0000x reference)
import functools

import jax
import jax.numpy as jnp
from jax import lax
from jax.experimental import pallas as pl
from jax.experimental.pallas import tpu as pltpu
from jax.experimental.pallas import tpu_sc as plsc

_B = 16384
_L = 50
_D = 64
_V = 1000000
_VMAIN = 999936
_VTAIL = _V - _VMAIN
_VPAD = _VMAIN + 128

_NT = 16
_BT = _B // _NT
_G = _BT // 128
_DPC = _D // 2
_LS = _L // 2
_NSLOT = 8

_CH_A = 62464
_CH_B = 62592
_N_A = 12


def _chunk(t):
    if t < _N_A:
        return t * _CH_A, _CH_A
    return _N_A * _CH_A + (t - _N_A) * _CH_B, _CH_B


def _sc_body(xi_hbm, mu_hbm, sig_hbm, mu_tail, sig_tail, mu_out, sig_out,
             idx_v, gbuf, feat_s, sem_g, sem_o):
    c = lax.axis_index("c")
    t = lax.axis_index("s")
    b0 = t * _BT

    for l in range(_L):
        pltpu.sync_copy(xi_hbm.at[l, pl.ds(t * _BT, _BT)],
                        idx_v.at[pl.ds(l * _BT, _BT)])

    def _load_main(row_view):
        for tt in range(_NT):
            off, n = _chunk(tt)

            @pl.when(t == tt)
            def _():
                pltpu.sync_copy(row_view.at[pl.ds(off, n)],
                                feat_s.at[pl.ds(off, n)])

    def _load_tail(tail_view):
        @pl.when(t == 0)
        def _():
            pltpu.sync_copy(tail_view, feat_s.at[pl.ds(_VMAIN, 128)])

    def _fire_gathers(l, slot):
        pltpu.async_copy(feat_s.at[idx_v.at[pl.ds(l * _BT, _BT)]],
                         gbuf.at[pl.ds(slot * _BT, _BT)], sem_g)

    def _wait_4k(sem):
        pltpu.make_async_copy(mu_tail.at[pl.ds(0, _BT)],
                              gbuf.at[pl.ds(0, _BT)], sem).wait()

    def _drain_out(n):
        def w_body(i, cc):
            _wait_4k(sem_o)
            return cc

        lax.fori_loop(0, n, w_body, 0)

    def _pipeline(n_l, l_of, write_out):
        _fire_gathers(l_of(0), 0)

        def l_body(l, cc):
            @pl.when(l >= _NSLOT - 1)
            def _():
                _wait_4k(sem_o)

            @pl.when(l < n_l - 1)
            def _():
                _fire_gathers(l_of(l + 1), lax.rem(l + 1, _NSLOT))

            _wait_4k(sem_g)
            write_out(l, lax.rem(l, _NSLOT))
            return cc

        lax.fori_loop(0, n_l, l_body, 0)
        _drain_out(_NSLOT - 1)

    def d_body(di, carry):
        d = c * _DPC + di
        _load_main(mu_hbm.at[d])
        _load_tail(mu_tail.at[pl.ds(d * 128, 128)])
        plsc.subcore_barrier()

        def mu_write(l, slot):
            pltpu.async_copy(gbuf.at[pl.ds(slot * _BT, _BT)],
                             mu_out.at[l, d, pl.ds(b0, _BT)], sem_o)

        _pipeline(_L, lambda l: l, mu_write)
        plsc.subcore_barrier()
        return carry

    lax.fori_loop(0, _DPC, d_body, 0)

    _load_main(sig_hbm)
    _load_tail(sig_tail)
    plsc.subcore_barrier()

    def sig_write(l, slot):
        pltpu.async_copy(gbuf.at[pl.ds(slot * _BT, _BT)],
                         sig_out.at[c * _LS + l, pl.ds(b0, _BT)], sem_o)

    _pipeline(_LS, lambda l: c * _LS + l, sig_write)
    plsc.subcore_barrier()


_sc_call = functools.partial(
    pl.kernel,
    out_type=(jax.ShapeDtypeStruct((_L, _D, _B), jnp.float32),
              jax.ShapeDtypeStruct((_L, _B), jnp.float32)),
    mesh=plsc.VectorSubcoreMesh(core_axis_name="c", subcore_axis_name="s"),
    scratch_types=[
        pltpu.VMEM((_L * _BT,), jnp.int32),
        pltpu.VMEM((_NSLOT * _BT,), jnp.float32),
        pltpu.VMEM_SHARED((_VPAD,), jnp.float32),
        pltpu.SemaphoreType.DMA,
        pltpu.SemaphoreType.DMA,
    ],
)(_sc_body)


def _softplus_body(s_ref, o_ref):
    v = s_ref[...]
    o_ref[...] = jnp.maximum(v, 0.0) + jnp.log1p(jnp.exp(-jnp.abs(v)))


def _softplus_tc(s2):
    return pl.pallas_call(
        _softplus_body,
        out_shape=jax.ShapeDtypeStruct(s2.shape, jnp.float32),
    )(s2)


def kernel(x, mu_table, sigma_table):
    x3 = x.T
    mu_t = mu_table.T
    sig_t = sigma_table.reshape(_V)
    mu_tail = jnp.pad(mu_t[:, _VMAIN:], ((0, 0), (0, 128 - _VTAIL))).reshape(-1)
    sig_tail = jnp.pad(sig_t[_VMAIN:], (0, 128 - _VTAIL))
    mu_p, sig_p = _sc_call(x3, mu_t, sig_t, mu_tail, sig_tail)
    sig_sp = _softplus_tc(sig_p)
    mu = jnp.transpose(mu_p, (2, 0, 1))
    sigma = jnp.transpose(sig_sp, (1, 0))[:, :, None]
    return (mu, sigma)


# speedup vs baseline: 1.0005x; 1.0005x over previous
import functools

import jax
import jax.numpy as jnp
from jax import lax
from jax.experimental import pallas as pl
from jax.experimental.pallas import tpu as pltpu
from jax.experimental.pallas import tpu_sc as plsc

_B = 16384
_L = 50
_D = 64
_V = 1000000
_VMAIN = 999936
_VTAIL = _V - _VMAIN
_VPAD = _VMAIN + 128

_NT = 16
_BT = _B // _NT
_G = _BT // 128
_DPC = _D // 2
_LS = _L // 2
_NSLOT = 8

_CH_A = 62464
_CH_B = 62592
_N_A = 12


def _chunk(t):
    if t < _N_A:
        return t * _CH_A, _CH_A
    return _N_A * _CH_A + (t - _N_A) * _CH_B, _CH_B


def _sc_body(xi_hbm, mu_hbm, sig_hbm, mu_tail, sig_tail, mu_out, sig_out,
             idx_v, gbuf, feat_s, sem_g, sem_o):
    c = lax.axis_index("c")
    t = lax.axis_index("s")
    b0 = t * _BT

    for l in range(_L):
        pltpu.sync_copy(xi_hbm.at[l, pl.ds(t * _BT, _BT)],
                        idx_v.at[pl.ds(l * _BT, _BT)])

    def _load_main(row_view):
        for tt in range(_NT):
            off, n = _chunk(tt)

            @pl.when(t == tt)
            def _():
                pltpu.sync_copy(row_view.at[pl.ds(off, n)],
                                feat_s.at[pl.ds(off, n)])

    def _load_tail(tail_view):
        @pl.when(t == 0)
        def _():
            pltpu.sync_copy(tail_view, feat_s.at[pl.ds(_VMAIN, 128)])

    def _fire_gathers(l, slot):
        for g in range(_G):
            pltpu.async_copy(
                feat_s.at[idx_v.at[pl.ds(l * _BT + g * 128, 128)]],
                gbuf.at[pl.ds(slot * _BT + g * 128, 128)], sem_g)

    def _wait_4k(sem):
        pltpu.make_async_copy(mu_tail.at[pl.ds(0, _BT)],
                              gbuf.at[pl.ds(0, _BT)], sem).wait()

    def _drain_out(n):
        def w_body(i, cc):
            _wait_4k(sem_o)
            return cc

        lax.fori_loop(0, n, w_body, 0)

    def _pipeline(n_l, l_of, write_out):
        _fire_gathers(l_of(0), 0)

        def l_body(l, cc):
            @pl.when(l >= _NSLOT - 1)
            def _():
                _wait_4k(sem_o)

            @pl.when(l < n_l - 1)
            def _():
                _fire_gathers(l_of(l + 1), lax.rem(l + 1, _NSLOT))

            _wait_4k(sem_g)
            write_out(l, lax.rem(l, _NSLOT))
            return cc

        lax.fori_loop(0, n_l, l_body, 0)
        _drain_out(_NSLOT - 1)

    def d_body(di, carry):
        d = c * _DPC + di
        _load_main(mu_hbm.at[d])
        _load_tail(mu_tail.at[pl.ds(d * 128, 128)])
        plsc.subcore_barrier()

        def mu_write(l, slot):
            pltpu.async_copy(gbuf.at[pl.ds(slot * _BT, _BT)],
                             mu_out.at[l, d, pl.ds(b0, _BT)], sem_o)

        _pipeline(_L, lambda l: l, mu_write)
        plsc.subcore_barrier()
        return carry

    lax.fori_loop(0, _DPC, d_body, 0)

    _load_main(sig_hbm)
    _load_tail(sig_tail)
    plsc.subcore_barrier()

    def sig_write(l, slot):
        pltpu.async_copy(gbuf.at[pl.ds(slot * _BT, _BT)],
                         sig_out.at[c * _LS + l, pl.ds(b0, _BT)], sem_o)

    _pipeline(_LS, lambda l: c * _LS + l, sig_write)
    plsc.subcore_barrier()


_sc_call = functools.partial(
    pl.kernel,
    out_type=(jax.ShapeDtypeStruct((_L, _D, _B), jnp.float32),
              jax.ShapeDtypeStruct((_L, _B), jnp.float32)),
    mesh=plsc.VectorSubcoreMesh(core_axis_name="c", subcore_axis_name="s"),
    scratch_types=[
        pltpu.VMEM((_L * _BT,), jnp.int32),
        pltpu.VMEM((_NSLOT * _BT,), jnp.float32),
        pltpu.VMEM_SHARED((_VPAD,), jnp.float32),
        pltpu.SemaphoreType.DMA,
        pltpu.SemaphoreType.DMA,
    ],
)(_sc_body)


def _softplus_body(s_ref, o_ref):
    v = s_ref[...]
    o_ref[...] = jnp.maximum(v, 0.0) + jnp.log1p(jnp.exp(-jnp.abs(v)))


def _softplus_tc(s2):
    return pl.pallas_call(
        _softplus_body,
        out_shape=jax.ShapeDtypeStruct(s2.shape, jnp.float32),
    )(s2)


def kernel(x, mu_table, sigma_table):
    x3 = x.T
    mu_t = mu_table.T
    sig_t = sigma_table.reshape(_V)
    mu_tail = jnp.pad(mu_t[:, _VMAIN:], ((0, 0), (0, 128 - _VTAIL))).reshape(-1)
    sig_tail = jnp.pad(sig_t[_VMAIN:], (0, 128 - _VTAIL))
    mu_p, sig_p = _sc_call(x3, mu_t, sig_t, mu_tail, sig_tail)
    sig_sp = _softplus_tc(sig_p)
    mu = jnp.transpose(mu_p, (2, 0, 1))
    sigma = jnp.transpose(sig_sp, (1, 0))[:, :, None]
    return (mu, sigma)


# speedup vs baseline: 1.0297x; 1.0292x over previous
import functools

import jax
import jax.numpy as jnp
from jax import lax
from jax.experimental import pallas as pl
from jax.experimental.pallas import tpu as pltpu
from jax.experimental.pallas import tpu_sc as plsc

_B = 16384
_L = 50
_D = 64
_V = 1000000
_VMAIN = 999936
_VTAIL = _V - _VMAIN
_VPAD = _VMAIN + 128

_NT = 16
_BT = _B // _NT
_G = _BT // 128
_DPC = _D // 2
_LS = _L // 2
_NSLOT = 8

_CH_A = 62464
_CH_B = 62592
_N_A = 12


def _chunk(t):
    if t < _N_A:
        return t * _CH_A, _CH_A
    return _N_A * _CH_A + (t - _N_A) * _CH_B, _CH_B


def _sc_body(xi_hbm, mu_hbm, sig_hbm, mu_tail, sig_tail, mu_out, sig_out,
             idx_v, gbuf, feat_s, sem_g, sem_o):
    c = lax.axis_index("c")
    t = lax.axis_index("s")
    b0 = t * _BT

    pltpu.sync_copy(xi_hbm.at[:, pl.ds(t * _BT, _BT)], idx_v)

    def _load_main(row_view):
        for tt in range(_NT):
            off, n = _chunk(tt)

            @pl.when(t == tt)
            def _():
                pltpu.sync_copy(row_view.at[pl.ds(off, n)],
                                feat_s.at[pl.ds(off, n)])

    def _load_tail(tail_view):
        @pl.when(t == 0)
        def _():
            pltpu.sync_copy(tail_view, feat_s.at[pl.ds(_VMAIN, 128)])

    def _fire_gathers(l, slot):
        for g in range(_G):
            pltpu.async_copy(
                feat_s.at[idx_v.at[l, pl.ds(g * 128, 128)]],
                gbuf.at[pl.ds(slot * _BT + g * 128, 128)], sem_g)

    def _wait_4k(sem):
        pltpu.make_async_copy(mu_tail.at[pl.ds(0, _BT)],
                              gbuf.at[pl.ds(0, _BT)], sem).wait()

    def _drain_out(n):
        def w_body(i, cc):
            _wait_4k(sem_o)
            return cc

        lax.fori_loop(0, n, w_body, 0)

    def _pipeline(n_l, l_of, write_out):
        _fire_gathers(l_of(0), 0)

        def l_body(l, cc):
            @pl.when(l >= _NSLOT - 1)
            def _():
                _wait_4k(sem_o)

            @pl.when(l < n_l - 1)
            def _():
                _fire_gathers(l_of(l + 1), lax.rem(l + 1, _NSLOT))

            _wait_4k(sem_g)
            write_out(l, lax.rem(l, _NSLOT))
            return cc

        lax.fori_loop(0, n_l, l_body, 0)
        _drain_out(_NSLOT - 1)

    def d_body(di, carry):
        d = c * _DPC + di
        _load_main(mu_hbm.at[d])
        _load_tail(mu_tail.at[pl.ds(d * 128, 128)])
        plsc.subcore_barrier()

        def mu_write(l, slot):
            pltpu.async_copy(gbuf.at[pl.ds(slot * _BT, _BT)],
                             mu_out.at[l, d, pl.ds(b0, _BT)], sem_o)

        _pipeline(_L, lambda l: l, mu_write)
        plsc.subcore_barrier()
        return carry

    lax.fori_loop(0, _DPC, d_body, 0)

    _load_main(sig_hbm)
    _load_tail(sig_tail)
    plsc.subcore_barrier()

    def sig_write(l, slot):
        pltpu.async_copy(gbuf.at[pl.ds(slot * _BT, _BT)],
                         sig_out.at[c * _LS + l, pl.ds(b0, _BT)], sem_o)

    _pipeline(_LS, lambda l: c * _LS + l, sig_write)
    plsc.subcore_barrier()


_sc_call = functools.partial(
    pl.kernel,
    out_type=(jax.ShapeDtypeStruct((_L, _D, _B), jnp.float32),
              jax.ShapeDtypeStruct((_L, _B), jnp.float32)),
    mesh=plsc.VectorSubcoreMesh(core_axis_name="c", subcore_axis_name="s"),
    scratch_types=[
        pltpu.VMEM((_L, _BT), jnp.int32),
        pltpu.VMEM((_NSLOT * _BT,), jnp.float32),
        pltpu.VMEM_SHARED((_VPAD,), jnp.float32),
        pltpu.SemaphoreType.DMA,
        pltpu.SemaphoreType.DMA,
    ],
)(_sc_body)


def _softplus_body(s_ref, o_ref):
    v = s_ref[...]
    o_ref[...] = jnp.maximum(v, 0.0) + jnp.log1p(jnp.exp(-jnp.abs(v)))


def _softplus_tc(s2):
    return pl.pallas_call(
        _softplus_body,
        out_shape=jax.ShapeDtypeStruct(s2.shape, jnp.float32),
    )(s2)


def kernel(x, mu_table, sigma_table):
    x3 = x.T
    mu_t = mu_table.T
    sig_t = sigma_table.reshape(_V)
    mu_tail = jnp.pad(mu_t[:, _VMAIN:], ((0, 0), (0, 128 - _VTAIL))).reshape(-1)
    sig_tail = jnp.pad(sig_t[_VMAIN:], (0, 128 - _VTAIL))
    mu_p, sig_p = _sc_call(x3, mu_t, sig_t, mu_tail, sig_tail)
    sig_sp = _softplus_tc(sig_p)
    mu = jnp.transpose(mu_p, (2, 0, 1))
    sigma = jnp.transpose(sig_sp, (1, 0))[:, :, None]
    return (mu, sigma)
